# Initial kernel scaffold; baseline (speedup 1.0000x reference)
import functools

import jax
import jax.numpy as jnp
from jax import lax
from jax.experimental import pallas as pl
from jax.experimental.pallas import tpu as pltpu, tpu_sc as plsc

HIDDEN = 64
LANES = 16
SLICES_PER_ROW = HIDDEN // LANES

NUM_CORES = 2
NUM_SUBCORES = 16
NW = NUM_CORES * NUM_SUBCORES

CHUNK = 512


def _emb_body(seq_hbm, pid_hbm, tok_hbm, pos_hbm, out_hbm,
              idx_v, pidx_v, trows, prows, sem, *, n_rows):
    per_w = n_rows // NW
    n_chunks = per_w // CHUNK
    wid = lax.axis_index("s") * NUM_CORES + lax.axis_index("c")
    base = wid * per_w

    def chunk_body(ci, carry):
        off = base + ci * CHUNK
        pltpu.sync_copy(seq_hbm.at[pl.ds(off, CHUNK)], idx_v)
        pltpu.sync_copy(pid_hbm.at[pl.ds(off, CHUNK)], pidx_v)
        cp_t = pltpu.async_copy(tok_hbm.at[idx_v], trows, sem)
        cp_p = pltpu.async_copy(pos_hbm.at[pidx_v], prows, sem)
        cp_t.wait()
        cp_p.wait()

        def add_body(r, carry2):
            for j in range(SLICES_PER_ROW):
                sl = pl.ds(j * LANES, LANES)
                trows[r, sl] = trows[r, sl] + prows[r, sl]
            return carry2

        lax.fori_loop(0, CHUNK, add_body, 0, unroll=False)
        pltpu.sync_copy(trows, out_hbm.at[pl.ds(off, CHUNK)])
        return carry

    lax.fori_loop(0, n_chunks, chunk_body, 0, unroll=False)


def kernel(sequence, position_ids, token_table, pos_table):
    batch, seq = sequence.shape
    n_rows = batch * seq
    seq_flat = sequence.reshape(n_rows).astype(jnp.int32)
    pid_flat = position_ids.reshape(n_rows).astype(jnp.int32)

    mesh = plsc.VectorSubcoreMesh(core_axis_name="c", subcore_axis_name="s",
                                  num_cores=NUM_CORES, num_subcores=NUM_SUBCORES)
    emb = functools.partial(
        pl.kernel,
        out_type=jax.ShapeDtypeStruct((n_rows, HIDDEN), jnp.float32),
        mesh=mesh,
        scratch_types=[
            pltpu.VMEM((CHUNK,), jnp.int32),
            pltpu.VMEM((CHUNK,), jnp.int32),
            pltpu.VMEM((CHUNK, HIDDEN), jnp.float32),
            pltpu.VMEM((CHUNK, HIDDEN), jnp.float32),
            pltpu.SemaphoreType.DMA,
        ],
    )(functools.partial(_emb_body, n_rows=n_rows))

    out = emb(seq_flat, pid_flat, token_table, pos_table)
    return out.reshape(batch, seq, HIDDEN)


# speedup vs baseline: 1.9952x; 1.9952x over previous
import functools

import jax
import jax.numpy as jnp
from jax import lax
from jax.experimental import pallas as pl
from jax.experimental.pallas import tpu as pltpu, tpu_sc as plsc

HIDDEN = 64
LANES = 16
SLICES_PER_ROW = HIDDEN // LANES

NUM_CORES = 2
NUM_SUBCORES = 16
NW = NUM_CORES * NUM_SUBCORES

CHUNK = 512


def _emb_body(seq_hbm, pid_hbm, tok_hbm, pos_hbm, out_hbm,
              idx_v, pidx_v, trows, prows, sem, *, n_rows):
    per_w = n_rows // NW
    n_chunks = per_w // CHUNK
    wid = lax.axis_index("s") * NUM_CORES + lax.axis_index("c")
    base = wid * per_w

    def chunk_body(ci, carry):
        off = base + ci * CHUNK
        pltpu.sync_copy(seq_hbm.at[pl.ds(off, CHUNK)], idx_v)
        pltpu.sync_copy(pid_hbm.at[pl.ds(off, CHUNK)], pidx_v)
        cp_t = pltpu.async_copy(tok_hbm.at[idx_v], trows, sem)
        cp_p = pltpu.async_copy(pos_hbm.at[pidx_v], prows, sem)
        cp_t.wait()
        cp_p.wait()

        def add_body(r, carry2):
            for j in range(SLICES_PER_ROW):
                sl = pl.ds(j * LANES, LANES)
                trows[r, sl] = trows[r, sl] + prows[r, sl]
            return carry2

        lax.fori_loop(0, CHUNK, add_body, 0, unroll=False)
        pltpu.sync_copy(trows, out_hbm.at[pl.ds(off, CHUNK)])
        return carry

    lax.fori_loop(0, n_chunks, chunk_body, 0, unroll=False)


def kernel(sequence, position_ids, token_table, pos_table):
    batch, seq = sequence.shape
    n_rows = batch * seq
    seq_flat = sequence.reshape(n_rows).astype(jnp.int32)
    pid_flat = position_ids.reshape(n_rows).astype(jnp.int32)

    mesh = plsc.VectorSubcoreMesh(core_axis_name="c", subcore_axis_name="s",
                                  num_cores=NUM_CORES, num_subcores=NUM_SUBCORES)
    emb = functools.partial(
        pl.kernel,
        out_type=jax.ShapeDtypeStruct((n_rows, HIDDEN), jnp.float32),
        mesh=mesh,
        scratch_types=[
            pltpu.VMEM((CHUNK,), jnp.int32),
            pltpu.VMEM((CHUNK,), jnp.int32),
            pltpu.VMEM((CHUNK, HIDDEN), jnp.float32),
            pltpu.VMEM((CHUNK, HIDDEN), jnp.float32),
            pltpu.SemaphoreType.DMA,
        ],
        compiler_params=pltpu.CompilerParams(use_tc_tiling_on_sc=False),
    )(functools.partial(_emb_body, n_rows=n_rows))

    out = emb(seq_flat, pid_flat, token_table, pos_table)
    return out.reshape(batch, seq, HIDDEN)


# speedup vs baseline: 2.0002x; 1.0025x over previous
import functools

import jax
import jax.numpy as jnp
from jax import lax
from jax.experimental import pallas as pl
from jax.experimental.pallas import tpu as pltpu, tpu_sc as plsc

HIDDEN = 64
NUM_CORES = 2
NUM_SUBCORES = 16
NW = NUM_CORES * NUM_SUBCORES

CHUNK = 800
NBUF = 2


def _emb_body(seq_hbm, pid_hbm, tok_hbm, pos_hbm, out_hbm, *refs, n_rows):
    idxs = refs[0:NBUF]
    pidxs = refs[NBUF:2 * NBUF]
    rows = refs[2 * NBUF:3 * NBUF]
    psem = refs[3 * NBUF:4 * NBUF]
    tsem = refs[4 * NBUF:5 * NBUF]
    osem = refs[5 * NBUF:6 * NBUF]

    per_w = n_rows // NW
    n_chunks = per_w // CHUNK
    wid = lax.axis_index("s") * NUM_CORES + lax.axis_index("c")
    base = wid * per_w

    out_cp = [None] * NBUF
    pos_cp = [None] * NBUF

    def stage(ci):
        b = ci % NBUF
        off = base + ci * CHUNK
        if out_cp[b] is not None:
            out_cp[b].wait()
        pltpu.sync_copy(seq_hbm.at[pl.ds(off, CHUNK)], idxs[b])
        pltpu.sync_copy(pid_hbm.at[pl.ds(off, CHUNK)], pidxs[b])
        pos_cp[b] = pltpu.async_copy(pos_hbm.at[pidxs[b]], rows[b], psem[b])

    stage(0)
    for ci in range(n_chunks):
        b = ci % NBUF
        if ci + 1 < n_chunks:
            stage(ci + 1)
        pos_cp[b].wait()
        tok_cp = pltpu.async_copy(tok_hbm.at[idxs[b]], rows[b], tsem[b], add=True)
        tok_cp.wait()
        out_cp[b] = pltpu.async_copy(
            rows[b], out_hbm.at[pl.ds(base + ci * CHUNK, CHUNK)], osem[b])
    for b in range(NBUF):
        if out_cp[b] is not None:
            out_cp[b].wait()


def kernel(sequence, position_ids, token_table, pos_table):
    batch, seq = sequence.shape
    n_rows = batch * seq
    seq_flat = sequence.reshape(n_rows).astype(jnp.int32)
    pid_flat = position_ids.reshape(n_rows).astype(jnp.int32)

    mesh = plsc.VectorSubcoreMesh(core_axis_name="c", subcore_axis_name="s",
                                  num_cores=NUM_CORES, num_subcores=NUM_SUBCORES)
    scratch = (
        [pltpu.VMEM((CHUNK,), jnp.int32) for _ in range(NBUF)]
        + [pltpu.VMEM((CHUNK,), jnp.int32) for _ in range(NBUF)]
        + [pltpu.VMEM((CHUNK, HIDDEN), jnp.float32) for _ in range(NBUF)]
        + [pltpu.SemaphoreType.DMA for _ in range(3 * NBUF)]
    )
    emb = functools.partial(
        pl.kernel,
        out_type=jax.ShapeDtypeStruct((n_rows, HIDDEN), jnp.float32),
        mesh=mesh,
        scratch_types=scratch,
        compiler_params=pltpu.CompilerParams(use_tc_tiling_on_sc=False),
    )(functools.partial(_emb_body, n_rows=n_rows))

    out = emb(seq_flat, pid_flat, token_table, pos_table)
    return out.reshape(batch, seq, HIDDEN)
